# Initial kernel scaffold; baseline (speedup 1.0000x reference)
import jax
import jax.numpy as jnp
from jax.experimental import pallas as pl


def kernel(x, weight, bias):
    raise NotImplementedError("write your pallas kernel here")


# speedup vs baseline: 1.1159x; 1.1159x over previous
import functools

import jax
import jax.numpy as jnp
from jax.experimental import pallas as pl
from jax.experimental.pallas import tpu as pltpu


def _conv3_kernel(x_ref, w_ref, b_ref, o_ref, s_ref, *, C, W, HW):
    pad = 2 * W
    s_ref[:, :pad] = jnp.zeros((C, pad), jnp.bfloat16)
    s_ref[:, pad + HW:] = jnp.zeros((C, pad), jnp.bfloat16)
    s_ref[:, pad:pad + HW] = x_ref[0].astype(jnp.bfloat16)

    col = jax.lax.rem(jax.lax.broadcasted_iota(jnp.int32, (C, HW), 1), W)
    m_left = (col != 0).astype(jnp.bfloat16)
    m_right = (col != W - 1).astype(jnp.bfloat16)

    acc = jnp.zeros((HW, w_ref.shape[2]), jnp.float32)
    for ki in range(3):
        for kj in range(3):
            off = ki * W + kj + (W - 1)
            sl = s_ref[:, off:off + HW]
            if kj == 0:
                sl = sl * m_left
            elif kj == 2:
                sl = sl * m_right
            acc = acc + jax.lax.dot_general(
                sl, w_ref[ki * 3 + kj],
                dimension_numbers=(((0,), (0,)), ((), ())),
                preferred_element_type=jnp.float32)
    o_ref[0] = acc + b_ref[...]


def kernel(x, weight, bias):
    B, C, H, W = x.shape
    O = weight.shape[0]
    HW = H * W

    xf = x.reshape(B, C, HW)
    w2 = jnp.transpose(weight.reshape(O, C, 9), (2, 1, 0)).astype(jnp.bfloat16)
    b2 = bias.reshape(1, O).astype(jnp.float32)

    kfn = functools.partial(_conv3_kernel, C=C, W=W, HW=HW)
    out = pl.pallas_call(
        kfn,
        out_shape=jax.ShapeDtypeStruct((B, HW, O), jnp.float32),
        grid=(B,),
        in_specs=[
            pl.BlockSpec((1, C, HW), lambda b: (b, 0, 0)),
            pl.BlockSpec((9, C, O), lambda b: (0, 0, 0)),
            pl.BlockSpec((1, O), lambda b: (0, 0)),
        ],
        out_specs=pl.BlockSpec((1, HW, O), lambda b: (b, 0, 0)),
        scratch_shapes=[pltpu.VMEM((C, HW + 4 * W), jnp.bfloat16)],
        compiler_params=pltpu.CompilerParams(
            dimension_semantics=("parallel",),
            vmem_limit_bytes=64 * 1024 * 1024,
        ),
    )(xf, w2, b2)

    return out.reshape(-1, O, H, W)


# speedup vs baseline: 1.4093x; 1.2629x over previous
import functools

import jax
import jax.numpy as jnp
from jax.experimental import pallas as pl
from jax.experimental.pallas import tpu as pltpu


def _conv3_kernel(x_ref, w_ref, b_ref, o_ref, s_ref, *, C, W, HW, NB):
    pad = 2 * W
    SW = HW + 2 * pad

    col = jax.lax.rem(jax.lax.broadcasted_iota(jnp.int32, (C, HW), 1), W)
    m_left = (col != 0).astype(jnp.bfloat16)
    m_right = (col != W - 1).astype(jnp.bfloat16)

    for n in range(NB):
        s_ref[n, :, :pad] = jnp.zeros((C, pad), jnp.bfloat16)
        s_ref[n, :, pad + HW:] = jnp.zeros((C, pad), jnp.bfloat16)
        s_ref[n, :, pad:pad + HW] = x_ref[n].astype(jnp.bfloat16)

    for n in range(NB):
        acc = jnp.zeros((HW, w_ref.shape[2]), jnp.float32)
        for ki in range(3):
            for kj in range(3):
                off = ki * W + kj + (W - 1)
                sl = s_ref[n, :, off:off + HW]
                if kj == 0:
                    sl = sl * m_left
                elif kj == 2:
                    sl = sl * m_right
                acc = acc + jax.lax.dot_general(
                    sl, w_ref[ki * 3 + kj],
                    dimension_numbers=(((0,), (0,)), ((), ())),
                    preferred_element_type=jnp.float32)
        o_ref[n] = acc + b_ref[...]


def kernel(x, weight, bias):
    B, C, H, W = x.shape
    O = weight.shape[0]
    HW = H * W

    xf = x.reshape(B, C, HW)
    w2 = jnp.transpose(weight.reshape(O, C, 9), (2, 1, 0)).astype(jnp.bfloat16)
    b2 = bias.reshape(1, O).astype(jnp.float32)

    NB = 4
    kfn = functools.partial(_conv3_kernel, C=C, W=W, HW=HW, NB=NB)
    out = pl.pallas_call(
        kfn,
        out_shape=jax.ShapeDtypeStruct((B, HW, O), jnp.float32),
        grid=(B // NB,),
        in_specs=[
            pl.BlockSpec((NB, C, HW), lambda b: (b, 0, 0)),
            pl.BlockSpec((9, C, O), lambda b: (0, 0, 0)),
            pl.BlockSpec((1, O), lambda b: (0, 0)),
        ],
        out_specs=pl.BlockSpec((NB, HW, O), lambda b: (b, 0, 0)),
        scratch_shapes=[pltpu.VMEM((NB, C, HW + 4 * W), jnp.bfloat16)],
        compiler_params=pltpu.CompilerParams(
            dimension_semantics=("parallel",),
            vmem_limit_bytes=64 * 1024 * 1024,
        ),
    )(xf, w2, b2)

    return out.reshape(-1, O, H, W)
